# Initial kernel scaffold; baseline (speedup 1.0000x reference)
import jax
import jax.numpy as jnp
from jax.experimental import pallas as pl


def kernel(predictions, targets):
    raise NotImplementedError("write your pallas kernel here")


# speedup vs baseline: 10.9641x; 10.9641x over previous
import jax
import jax.numpy as jnp
from jax.experimental import pallas as pl
from jax.experimental.pallas import tpu as pltpu

Z, X, Y = 4, 32, 32
ZY = Z * Y
PAD_X = 4
LANES = 384
LANE0 = 128
ROWS = X + 2 * PAD_X
NP = 8
CONF_THR = 0.7
T_THR = 0.5
SCALE = 1.4
D_ELEM = (0.74, 0.528)
SZ = 3.0 / 4.0
SXY = 25.0 / 32.0
NEG = -1e9
MATCH_R2 = 0.25


def _nms_kernel(p_ref, t_ref, o_ref, ch_scr, r_scr, s_scr, r2_scr, s2_scr):
    lane = jax.lax.broadcasted_iota(jnp.int32, (ROWS, LANES), 1)
    row = jax.lax.broadcasted_iota(jnp.int32, (ROWS, LANES), 0)
    zzf = ((lane + LANE0) // Y - (2 * LANE0) // Y).astype(jnp.float32)
    yyf = ((lane + LANE0) % Y).astype(jnp.float32)
    xxf = (row - PAD_X).astype(jnp.float32)
    lane_id = jax.lax.broadcasted_iota(jnp.int32, (NP, 128), 1)
    pair_id = jax.lax.broadcasted_iota(jnp.int32, (NP, 1, 1), 0)
    thrO = D_ELEM[0] * SCALE
    thrH = D_ELEM[1] * SCALE
    thr2v = jnp.where(pair_id % 2 == 0, jnp.float32(thrO * thrO),
                      jnp.float32(thrH * thrH))

    planes = [
        p_ref[:, 0],
        (p_ref[:, 1] + zzf[None]) * SZ,
        (p_ref[:, 2] + xxf[None]) * SXY,
        (p_ref[:, 3] + yyf[None]) * SXY,
        t_ref[:, 0],
        (t_ref[:, 1] + zzf[None]) * SZ,
        (t_ref[:, 2] + xxf[None]) * SXY,
        (t_ref[:, 3] + yyf[None]) * SXY,
    ]
    for c, plane in enumerate(planes):
        for dxi in (range(5) if c < 4 else range(1, 4)):
            d = dxi - 2
            ch_scr[c, dxi, :, PAD_X:PAD_X + X, :] = \
                plane[:, PAD_X + d:PAD_X + d + X, :]
    r_scr[...] = jnp.zeros((NP, ROWS, LANES), jnp.float32)
    s_scr[...] = jnp.zeros((NP, ROWS, LANES), jnp.float32)

    def C(c):
        return planes[c][:, PAD_X:PAD_X + X, LANE0:LANE0 + ZY]

    conf, cz, cx, cy = C(0), C(1), C(2), C(3)
    tconf, tcz, tcx, tcy = C(4), C(5), C(6), C(7)
    valid = conf > CONF_THR
    validf = valid.astype(jnp.float32)
    t_valid = tconf > T_THR

    shifts25 = [(dz, dy) for dz in range(-2, 3) for dy in range(-2, 3)]

    def nms_terms(dxi, use_restrain):
        acc = jnp.zeros((NP, X, ZY), jnp.float32)
        for dz, dy in shifts25:
            s = dz * Y + dy

            def SH(c):
                return ch_scr[c, dxi, :, PAD_X:PAD_X + X,
                              LANE0 + s:LANE0 + s + ZY]

            cs = SH(0)
            d2 = (SH(1) - cz) ** 2 + (SH(2) - cx) ** 2 + (SH(3) - cy) ** 2
            tie = (dz * (X * Y) + dy + (dxi - 2) * Y) < 0
            rank = (cs > conf) | jnp.logical_and(cs == conf, tie)
            a = (d2 < thr2v) & rank & (cs > CONF_THR)
            if use_restrain:
                rs = r2_scr[dxi, :, PAD_X:PAD_X + X, LANE0 + s:LANE0 + s + ZY]
                a = a & (rs != 0)
            acc = acc + a.astype(jnp.float32)
        return acc

    restrain = jax.lax.fori_loop(
        0, 5, lambda i, c: c + nms_terms(i, False),
        jnp.zeros((NP, X, ZY), jnp.float32)) * validf
    r_scr[:, PAD_X:PAD_X + X, LANE0:LANE0 + ZY] = restrain
    for dxi in range(5):
        d = dxi - 2
        r2_scr[dxi, :, PAD_X:PAD_X + X, :] = r_scr[:, PAD_X + d:PAD_X + d + X, :]

    sub = jax.lax.fori_loop(
        0, 5, lambda i, c: c + nms_terms(i, True),
        jnp.zeros((NP, X, ZY), jnp.float32))
    p_sel = ((restrain - sub * validf) == 0) & valid
    s_scr[:, PAD_X:PAD_X + X, LANE0:LANE0 + ZY] = p_sel.astype(jnp.float32)
    for dxi in range(1, 4):
        d = dxi - 2
        s2_scr[dxi, :, PAD_X:PAD_X + X, :] = s_scr[:, PAD_X + d:PAD_X + d + X, :]

    def match_terms(i, carry):
        mt, mp = carry
        dxi = i + 1
        for dz, dy in [(dz, dy) for dz in range(-1, 2) for dy in range(-1, 2)]:
            s = dz * Y + dy

            def SH(c):
                return ch_scr[c, dxi, :, PAD_X:PAD_X + X,
                              LANE0 + s:LANE0 + s + ZY]

            d2 = (SH(1) - tcz) ** 2 + (SH(2) - tcx) ** 2 + (SH(3) - tcy) ** 2
            ps = s2_scr[dxi, :, PAD_X:PAD_X + X, LANE0 + s:LANE0 + s + ZY]
            mt = jnp.maximum(mt, ((d2 < MATCH_R2) & (ps != 0))
                             .astype(jnp.float32))
            d2b = (SH(5) - cz) ** 2 + (SH(6) - cx) ** 2 + (SH(7) - cy) ** 2
            mp = jnp.maximum(mp, ((d2b < MATCH_R2) & (SH(4) > T_THR))
                             .astype(jnp.float32))
        return mt, mp

    mt_f, mp_f = jax.lax.fori_loop(
        0, 3, match_terms,
        (jnp.zeros((NP, X, ZY), jnp.float32), jnp.zeros((NP, X, ZY), jnp.float32)))
    matched_t = mt_f != 0
    matched_p = mp_f != 0

    def count(m):
        return jnp.sum(jnp.sum(m.astype(jnp.float32), axis=2), axis=1,
                       keepdims=True)

    tp = count(matched_t & t_valid)
    fn = count((~matched_t) & t_valid)
    fp = count(p_sel & (~matched_p))
    o_ref[...] = jnp.where(lane_id == 0, tp,
                           jnp.where(lane_id == 1, fp,
                                     jnp.where(lane_id == 2, fn, 0.0)))


def _prep(x):
    B = x.shape[0]
    v = x.reshape(B, 2, 4, Z, X, Y)
    v = jnp.transpose(v, (0, 1, 2, 4, 3, 5)).reshape(NP, 4, X, ZY)
    return jnp.pad(v, ((0, 0), (0, 0), (PAD_X, PAD_X), (LANE0, LANE0)),
                   constant_values=NEG)


@jax.jit
def kernel(predictions, targets):
    p = _prep(predictions)
    t = _prep(targets)
    out = pl.pallas_call(
        _nms_kernel,
        out_shape=jax.ShapeDtypeStruct((NP, 128), jnp.float32),
        scratch_shapes=[pltpu.VMEM((8, 5, NP, ROWS, LANES), jnp.float32),
                        pltpu.VMEM((NP, ROWS, LANES), jnp.float32),
                        pltpu.VMEM((NP, ROWS, LANES), jnp.float32),
                        pltpu.VMEM((5, NP, ROWS, LANES), jnp.float32),
                        pltpu.VMEM((5, NP, ROWS, LANES), jnp.float32)],
    )(p, t)
    return out[:, :3].reshape(4, 2, 3)


# speedup vs baseline: 13.8058x; 1.2592x over previous
import jax
import jax.numpy as jnp
from jax.experimental import pallas as pl
from jax.experimental.pallas import tpu as pltpu

Z, X, Y = 4, 32, 32
ZY = Z * Y
PAD_X = 4
LANES = 384
LANE0 = 128
ROWS = X + 2 * PAD_X
NP = 8
CONF_THR = 0.7
T_THR = 0.5
SCALE = 1.4
D_ELEM = (0.74, 0.528)
SZ = 3.0 / 4.0
SXY = 25.0 / 32.0
NEG = -1e9
MATCH_R2 = 0.25
SHIFTS = {1: [(dz, dy) for dz in range(-1, 2) for dy in range(-1, 2)],
          2: [(dz, dy) for dz in range(-2, 3) for dy in range(-2, 3)]}


def _nms_kernel(p_ref, t_ref, o_ref, ch_scr, r_scr, s_scr, r2_scr, s2_scr):
    lane = jax.lax.broadcasted_iota(jnp.int32, (ROWS, LANES), 1)
    row = jax.lax.broadcasted_iota(jnp.int32, (ROWS, LANES), 0)
    zzf = ((lane + LANE0) // Y - (2 * LANE0) // Y).astype(jnp.float32)
    yyf = ((lane + LANE0) % Y).astype(jnp.float32)
    xxf = (row - PAD_X).astype(jnp.float32)
    lane_id = jax.lax.broadcasted_iota(jnp.int32, (4, 128), 1)

    planes = [
        p_ref[:, 0],
        (p_ref[:, 1] + zzf[None]) * SZ,
        (p_ref[:, 2] + xxf[None]) * SXY,
        (p_ref[:, 3] + yyf[None]) * SXY,
        t_ref[:, 0],
        (t_ref[:, 1] + zzf[None]) * SZ,
        (t_ref[:, 2] + xxf[None]) * SXY,
        (t_ref[:, 3] + yyf[None]) * SXY,
    ]
    for c, plane in enumerate(planes):
        for dxi in (range(5) if c < 4 else range(1, 4)):
            d = dxi - 2
            ch_scr[c, dxi, :, PAD_X:PAD_X + X, :] = \
                plane[:, PAD_X + d:PAD_X + d + X, :]
    r_scr[...] = jnp.zeros((NP, ROWS, LANES), jnp.float32)
    s_scr[...] = jnp.zeros((NP, ROWS, LANES), jnp.float32)

    CTR = (slice(None), slice(PAD_X, PAD_X + X), slice(LANE0, LANE0 + ZY))
    ctrs = {}
    for e, g in ((0, slice(0, 4)), (1, slice(4, 8))):
        ctrs[e] = [planes[c][g][CTR] for c in range(8)]

    def nms_terms(e, g, reach, thr2, dxi, use_restrain):
        conf, cz, cx, cy = ctrs[e][:4]
        acc = jnp.zeros((4, X, ZY), jnp.float32)
        for dz, dy in SHIFTS[reach]:
            s = dz * Y + dy

            def SH(c):
                return ch_scr[c, dxi, g, PAD_X:PAD_X + X,
                              LANE0 + s:LANE0 + s + ZY]

            cs = SH(0)
            d2 = (SH(1) - cz) ** 2 + (SH(2) - cx) ** 2 + (SH(3) - cy) ** 2
            tie = (dz * (X * Y) + dy + (dxi - 2) * Y) < 0
            rank = (cs > conf) | jnp.logical_and(cs == conf, tie)
            a = (d2 < thr2) & rank & (cs > CONF_THR)
            if use_restrain:
                rs = r2_scr[dxi, g, PAD_X:PAD_X + X, LANE0 + s:LANE0 + s + ZY]
                a = a & (rs != 0)
            acc = acc + a.astype(jnp.float32)
        return acc

    groups = []
    for e, g, reach in ((0, slice(0, 4), 2), (1, slice(4, 8), 1)):
        thr = D_ELEM[e] * SCALE
        thr2 = jnp.float32(float(thr) * float(thr))
        lo, hi = (0, 5) if reach == 2 else (1, 4)
        conf = ctrs[e][0]
        valid = conf > CONF_THR
        validf = valid.astype(jnp.float32)
        restrain = jax.lax.fori_loop(
            lo, hi, lambda i, c: c + nms_terms(e, g, reach, thr2, i, False),
            jnp.zeros((4, X, ZY), jnp.float32)) * validf
        r_scr[g, PAD_X:PAD_X + X, LANE0:LANE0 + ZY] = restrain
        groups.append((e, g, reach, thr2, lo, hi, valid, validf, restrain))

    for dxi in range(5):
        d = dxi - 2
        r2_scr[dxi, :, PAD_X:PAD_X + X, :] = r_scr[:, PAD_X + d:PAD_X + d + X, :]

    sels = []
    for e, g, reach, thr2, lo, hi, valid, validf, restrain in groups:
        sub = jax.lax.fori_loop(
            lo, hi, lambda i, c: c + nms_terms(e, g, reach, thr2, i, True),
            jnp.zeros((4, X, ZY), jnp.float32))
        p_sel = ((restrain - sub * validf) == 0) & valid
        s_scr[g, PAD_X:PAD_X + X, LANE0:LANE0 + ZY] = p_sel.astype(jnp.float32)
        sels.append(p_sel)

    for dxi in range(1, 4):
        d = dxi - 2
        s2_scr[dxi, :, PAD_X:PAD_X + X, :] = s_scr[:, PAD_X + d:PAD_X + d + X, :]

    def count(m):
        return jnp.sum(jnp.sum(m.astype(jnp.float32), axis=2), axis=1,
                       keepdims=True)

    for (e, g, reach, thr2, lo, hi, valid, validf, restrain), p_sel in zip(
            groups, sels):
        cz, cx, cy = ctrs[e][1:4]
        tconf, tcz, tcx, tcy = ctrs[e][4:8]
        t_valid = tconf > T_THR

        def match_terms(i, carry):
            mt, mp = carry
            dxi = i + 1
            for dz, dy in SHIFTS[1]:
                s = dz * Y + dy

                def SH(c):
                    return ch_scr[c, dxi, g, PAD_X:PAD_X + X,
                                  LANE0 + s:LANE0 + s + ZY]

                d2 = ((SH(1) - tcz) ** 2 + (SH(2) - tcx) ** 2
                      + (SH(3) - tcy) ** 2)
                ps = s2_scr[dxi, g, PAD_X:PAD_X + X, LANE0 + s:LANE0 + s + ZY]
                mt = jnp.maximum(mt, ((d2 < MATCH_R2) & (ps != 0))
                                 .astype(jnp.float32))
                d2b = ((SH(5) - cz) ** 2 + (SH(6) - cx) ** 2
                       + (SH(7) - cy) ** 2)
                mp = jnp.maximum(mp, ((d2b < MATCH_R2) & (SH(4) > T_THR))
                                 .astype(jnp.float32))
            return mt, mp

        mt_f, mp_f = jax.lax.fori_loop(
            0, 3, match_terms,
            (jnp.zeros((4, X, ZY), jnp.float32),
             jnp.zeros((4, X, ZY), jnp.float32)))
        matched_t = (mt_f != 0) & t_valid
        tp = count(matched_t)
        fn = count((mt_f == 0) & t_valid)
        fp = count(p_sel & (mp_f == 0))
        o_ref[g] = jnp.where(lane_id == 0, tp,
                             jnp.where(lane_id == 1, fp,
                                       jnp.where(lane_id == 2, fn, 0.0)))


def _prep(x):
    B = x.shape[0]
    v = x.reshape(B, 2, 4, Z, X, Y)
    v = jnp.transpose(v, (1, 0, 2, 4, 3, 5)).reshape(NP, 4, X, ZY)
    return jnp.pad(v, ((0, 0), (0, 0), (PAD_X, PAD_X), (LANE0, LANE0)),
                   constant_values=NEG)


@jax.jit
def kernel(predictions, targets):
    p = _prep(predictions)
    t = _prep(targets)
    out = pl.pallas_call(
        _nms_kernel,
        out_shape=jax.ShapeDtypeStruct((NP, 128), jnp.float32),
        scratch_shapes=[pltpu.VMEM((8, 5, NP, ROWS, LANES), jnp.float32),
                        pltpu.VMEM((NP, ROWS, LANES), jnp.float32),
                        pltpu.VMEM((NP, ROWS, LANES), jnp.float32),
                        pltpu.VMEM((5, NP, ROWS, LANES), jnp.float32),
                        pltpu.VMEM((5, NP, ROWS, LANES), jnp.float32)],
    )(p, t)
    return jnp.transpose(out[:, :3].reshape(2, 4, 3), (1, 0, 2))


# speedup vs baseline: 18.1342x; 1.3135x over previous
import jax
import jax.numpy as jnp
from jax.experimental import pallas as pl
from jax.experimental.pallas import tpu as pltpu

Z, X, Y = 4, 32, 32
ZY = Z * Y
PAD_X = 4
LANES = 384
LANE0 = 128
ROWS = X + 2 * PAD_X
NP = 8
CONF_THR = 0.7
T_THR = 0.5
SCALE = 1.4
D_ELEM = (0.74, 0.528)
SZ = 3.0 / 4.0
SXY = 25.0 / 32.0
NEG = -1e9
MATCH_R2 = 0.25
SHIFTS = {1: [(dz, dy) for dz in range(-1, 2) for dy in range(-1, 2)],
          2: [(dz, dy) for dz in range(-2, 3) for dy in range(-2, 3)]}


def _nms_kernel(p_ref, t_ref, o_ref, ch_scr, r_scr, s_scr, r2_scr, s2_scr):
    lane = jax.lax.broadcasted_iota(jnp.int32, (ROWS, LANES), 1)
    row = jax.lax.broadcasted_iota(jnp.int32, (ROWS, LANES), 0)
    zzf = ((lane + LANE0) // Y - (2 * LANE0) // Y).astype(jnp.float32)
    yyf = ((lane + LANE0) % Y).astype(jnp.float32)
    xxf = (row - PAD_X).astype(jnp.float32)
    lane_id = jax.lax.broadcasted_iota(jnp.int32, (4, 128), 1)

    planes = [
        p_ref[:, 0],
        (p_ref[:, 1] + zzf[None]) * SZ,
        (p_ref[:, 2] + xxf[None]) * SXY,
        (p_ref[:, 3] + yyf[None]) * SXY,
        t_ref[:, 0],
        (t_ref[:, 1] + zzf[None]) * SZ,
        (t_ref[:, 2] + xxf[None]) * SXY,
        (t_ref[:, 3] + yyf[None]) * SXY,
    ]
    for c, plane in enumerate(planes):
        for dxi in (range(5) if c < 4 else range(1, 4)):
            d = dxi - 2
            ch_scr[c, dxi, :, PAD_X:PAD_X + X, :] = \
                plane[:, PAD_X + d:PAD_X + d + X, :]
    r_scr[...] = jnp.zeros((NP, ROWS, LANES), jnp.float32)
    s_scr[...] = jnp.zeros((NP, ROWS, LANES), jnp.float32)

    CTR = (slice(None), slice(PAD_X, PAD_X + X), slice(LANE0, LANE0 + ZY))
    ctrs = {}
    for e, g in ((0, slice(0, 4)), (1, slice(4, 8))):
        ctrs[e] = [planes[c][g][CTR] for c in range(8)]

    def nms_terms(e, g, thr2, dxi, shifts, use_restrain):
        conf, cz, cx, cy = ctrs[e][:4]
        acc = jnp.zeros((4, X, ZY), jnp.float32)
        for dz, dy in shifts:
            s = dz * Y + dy

            def SH(c):
                return ch_scr[c, dxi, g, PAD_X:PAD_X + X,
                              LANE0 + s:LANE0 + s + ZY]

            cs = SH(0)
            d2 = (SH(1) - cz) ** 2 + (SH(2) - cx) ** 2 + (SH(3) - cy) ** 2
            if dz != 0:
                rank = (cs >= conf) if dz < 0 else (cs > conf)
            else:
                tie = ((dxi - 2) * Y + dy) < 0
                rank = (cs > conf) | jnp.logical_and(cs == conf, tie)
            a = (d2 < thr2) & rank & (cs > CONF_THR)
            if use_restrain:
                rs = r2_scr[dxi, g, PAD_X:PAD_X + X, LANE0 + s:LANE0 + s + ZY]
                a = a & (rs != 0)
            acc = acc + a.astype(jnp.float32)
        return acc

    S33 = SHIFTS[1]
    S21 = [(dz, dy) for dz, dy in SHIFTS[2] if abs(dz) < 2 or abs(dy) < 2]
    PLANS = {0: [(1, 4, lambda i: i, S21), (0, 2, lambda i: i * 4, S33)],
             1: [(1, 4, lambda i: i, S33)]}

    def nms_pass(e, g, thr2, use_restrain):
        total = jnp.zeros((4, X, ZY), jnp.float32)
        for lo, hi, dxi_of, shifts in PLANS[e]:
            total = jax.lax.fori_loop(
                lo, hi,
                lambda i, c: c + nms_terms(e, g, thr2, dxi_of(i), shifts,
                                           use_restrain),
                total)
        return total

    groups = []
    for e, g in ((0, slice(0, 4)), (1, slice(4, 8))):
        thr = D_ELEM[e] * SCALE
        thr2 = jnp.float32(float(thr) * float(thr))
        conf = ctrs[e][0]
        valid = conf > CONF_THR
        validf = valid.astype(jnp.float32)
        restrain = nms_pass(e, g, thr2, False) * validf
        r_scr[g, PAD_X:PAD_X + X, LANE0:LANE0 + ZY] = restrain
        groups.append((e, g, thr2, valid, validf, restrain))

    for dxi in range(5):
        d = dxi - 2
        r2_scr[dxi, :, PAD_X:PAD_X + X, :] = r_scr[:, PAD_X + d:PAD_X + d + X, :]

    sels = []
    for e, g, thr2, valid, validf, restrain in groups:
        sub = nms_pass(e, g, thr2, True)
        p_sel = ((restrain - sub * validf) == 0) & valid
        s_scr[g, PAD_X:PAD_X + X, LANE0:LANE0 + ZY] = p_sel.astype(jnp.float32)
        sels.append(p_sel)

    for dxi in range(1, 4):
        d = dxi - 2
        s2_scr[dxi, :, PAD_X:PAD_X + X, :] = s_scr[:, PAD_X + d:PAD_X + d + X, :]

    def count(m):
        return jnp.sum(jnp.sum(m.astype(jnp.float32), axis=2), axis=1,
                       keepdims=True)

    for (e, g, thr2, valid, validf, restrain), p_sel in zip(groups, sels):
        cz, cx, cy = ctrs[e][1:4]
        tconf, tcz, tcx, tcy = ctrs[e][4:8]
        t_valid = tconf > T_THR

        def match_terms(i, carry):
            mt, mp = carry
            dxi = i + 1
            for dz, dy in SHIFTS[1]:
                s = dz * Y + dy

                def SH(c):
                    return ch_scr[c, dxi, g, PAD_X:PAD_X + X,
                                  LANE0 + s:LANE0 + s + ZY]

                d2 = ((SH(1) - tcz) ** 2 + (SH(2) - tcx) ** 2
                      + (SH(3) - tcy) ** 2)
                ps = s2_scr[dxi, g, PAD_X:PAD_X + X, LANE0 + s:LANE0 + s + ZY]
                mt = jnp.maximum(mt, ((d2 < MATCH_R2) & (ps != 0))
                                 .astype(jnp.float32))
                d2b = ((SH(5) - cz) ** 2 + (SH(6) - cx) ** 2
                       + (SH(7) - cy) ** 2)
                mp = jnp.maximum(mp, ((d2b < MATCH_R2) & (SH(4) > T_THR))
                                 .astype(jnp.float32))
            return mt, mp

        mt_f, mp_f = jax.lax.fori_loop(
            0, 3, match_terms,
            (jnp.zeros((4, X, ZY), jnp.float32),
             jnp.zeros((4, X, ZY), jnp.float32)))
        matched_t = (mt_f != 0) & t_valid
        tp = count(matched_t)
        fn = count((mt_f == 0) & t_valid)
        fp = count(p_sel & (mp_f == 0))
        o_ref[g] = jnp.where(lane_id == 0, tp,
                             jnp.where(lane_id == 1, fp,
                                       jnp.where(lane_id == 2, fn, 0.0)))


def _prep(x):
    B = x.shape[0]
    v = x.reshape(B, 2, 4, Z, X, Y)
    v = jnp.transpose(v, (1, 0, 2, 4, 3, 5)).reshape(NP, 4, X, ZY)
    return jnp.pad(v, ((0, 0), (0, 0), (PAD_X, PAD_X), (LANE0, LANE0)),
                   constant_values=NEG)


@jax.jit
def kernel(predictions, targets):
    p = _prep(predictions)
    t = _prep(targets)
    out = pl.pallas_call(
        _nms_kernel,
        out_shape=jax.ShapeDtypeStruct((NP, 128), jnp.float32),
        scratch_shapes=[pltpu.VMEM((8, 5, NP, ROWS, LANES), jnp.float32),
                        pltpu.VMEM((NP, ROWS, LANES), jnp.float32),
                        pltpu.VMEM((NP, ROWS, LANES), jnp.float32),
                        pltpu.VMEM((5, NP, ROWS, LANES), jnp.float32),
                        pltpu.VMEM((5, NP, ROWS, LANES), jnp.float32)],
    )(p, t)
    return jnp.transpose(out[:, :3].reshape(2, 4, 3), (1, 0, 2))


# speedup vs baseline: 22.3786x; 1.2341x over previous
import jax
import jax.numpy as jnp
from jax.experimental import pallas as pl
from jax.experimental.pallas import tpu as pltpu

Z, X, Y = 4, 32, 32
ZY = Z * Y
PAD_X = 4
LANES = 384
LANE0 = 128
ROWS = X + 2 * PAD_X
NP = 8
CONF_THR = 0.7
T_THR = 0.5
SCALE = 1.4
D_ELEM = (0.74, 0.528)
SZ = 3.0 / 4.0
SXY = 25.0 / 32.0
NEG = -1e9
MATCH_R2 = 0.25
SHIFTS = {1: [(dz, dy) for dz in range(-1, 2) for dy in range(-1, 2)],
          2: [(dz, dy) for dz in range(-2, 3) for dy in range(-2, 3)]}


def _nms_kernel(p_ref, t_ref, o_ref, ch_scr, r_scr, s_scr, r2_scr, s2_scr,
                b_scr):
    lane = jax.lax.broadcasted_iota(jnp.int32, (ROWS, LANES), 1)
    row = jax.lax.broadcasted_iota(jnp.int32, (ROWS, LANES), 0)
    zzf = ((lane + LANE0) // Y - (2 * LANE0) // Y).astype(jnp.float32)
    yyf = ((lane + LANE0) % Y).astype(jnp.float32)
    xxf = (row - PAD_X).astype(jnp.float32)
    lane_id = jax.lax.broadcasted_iota(jnp.int32, (4, 128), 1)

    planes = [
        p_ref[:, 0],
        (p_ref[:, 1] + zzf[None]) * SZ,
        (p_ref[:, 2] + xxf[None]) * SXY,
        (p_ref[:, 3] + yyf[None]) * SXY,
        t_ref[:, 0],
        (t_ref[:, 1] + zzf[None]) * SZ,
        (t_ref[:, 2] + xxf[None]) * SXY,
        (t_ref[:, 3] + yyf[None]) * SXY,
    ]
    for c, plane in enumerate(planes):
        for dxi in (range(5) if c < 4 else range(1, 4)):
            d = dxi - 2
            ch_scr[c, dxi, :, PAD_X:PAD_X + X, :] = \
                plane[:, PAD_X + d:PAD_X + d + X, :]
    r_scr[...] = jnp.zeros((NP, ROWS, LANES), jnp.float32)
    s_scr[...] = jnp.zeros((NP, ROWS, LANES), jnp.float32)

    CTR = (slice(None), slice(PAD_X, PAD_X + X), slice(LANE0, LANE0 + ZY))
    ctrs = {}
    for e, g in ((0, slice(0, 4)), (1, slice(4, 8))):
        ctrs[e] = [planes[c][g][CTR] for c in range(8)]

    def nms_trip(e, g, thr2, dxi, shifts, phase2):
        conf, cz, cx, cy = ctrs[e][:4]
        m = jnp.zeros((4, X, ZY), jnp.float32)
        if phase2:
            bits = b_scr[dxi, g]
        else:
            bits = jnp.zeros((4, X, ZY), jnp.int32)
        for j, (dz, dy) in enumerate(shifts):
            s = dz * Y + dy
            if phase2:
                rs = r2_scr[dxi, g, PAD_X:PAD_X + X, LANE0 + s:LANE0 + s + ZY]
                term = ((bits & (1 << j)) != 0) & (rs == 0)
            else:
                def SH(c):
                    return ch_scr[c, dxi, g, PAD_X:PAD_X + X,
                                  LANE0 + s:LANE0 + s + ZY]

                cs = SH(0)
                d2 = ((SH(1) - cz) ** 2 + (SH(2) - cx) ** 2
                      + (SH(3) - cy) ** 2)
                if dz != 0:
                    rank = (cs >= conf) if dz < 0 else (cs > conf)
                else:
                    tie = ((dxi - 2) * Y + dy) < 0
                    rank = (cs > conf) | jnp.logical_and(cs == conf, tie)
                term = (d2 < thr2) & rank & (cs > CONF_THR)
                bits = bits | jnp.where(term, jnp.int32(1 << j), jnp.int32(0))
            m = jnp.maximum(m, term.astype(jnp.float32))
        if not phase2:
            b_scr[dxi, g] = bits
        return m

    S33 = SHIFTS[1]
    S21 = [(dz, dy) for dz, dy in SHIFTS[2] if abs(dz) < 2 or abs(dy) < 2]
    PLANS = {0: [(1, 4, lambda i: i, S21), (0, 2, lambda i: i * 4, S33)],
             1: [(1, 4, lambda i: i, S33)]}

    def nms_pass(e, g, thr2, phase2):
        total = jnp.zeros((4, X, ZY), jnp.float32)
        for lo, hi, dxi_of, shifts in PLANS[e]:
            total = jax.lax.fori_loop(
                lo, hi,
                lambda i, c: jnp.maximum(
                    c, nms_trip(e, g, thr2, dxi_of(i), shifts, phase2)),
                total)
        return total

    groups = []
    for e, g in ((0, slice(0, 4)), (1, slice(4, 8))):
        thr = D_ELEM[e] * SCALE
        thr2 = jnp.float32(float(thr) * float(thr))
        conf = ctrs[e][0]
        valid = conf > CONF_THR
        validf = valid.astype(jnp.float32)
        restrain = nms_pass(e, g, thr2, False) * validf
        r_scr[g, PAD_X:PAD_X + X, LANE0:LANE0 + ZY] = restrain
        groups.append((e, g, thr2, valid, validf, restrain))

    for dxi in range(5):
        d = dxi - 2
        r2_scr[dxi, :, PAD_X:PAD_X + X, :] = r_scr[:, PAD_X + d:PAD_X + d + X, :]

    sels = []
    for e, g, thr2, valid, validf, restrain in groups:
        bad = nms_pass(e, g, thr2, True)
        p_sel = (bad == 0) & valid
        s_scr[g, PAD_X:PAD_X + X, LANE0:LANE0 + ZY] = p_sel.astype(jnp.float32)
        sels.append(p_sel)

    for dxi in range(1, 4):
        d = dxi - 2
        s2_scr[dxi, :, PAD_X:PAD_X + X, :] = s_scr[:, PAD_X + d:PAD_X + d + X, :]

    def count(m):
        return jnp.sum(jnp.sum(m.astype(jnp.float32), axis=2), axis=1,
                       keepdims=True)

    for (e, g, thr2, valid, validf, restrain), p_sel in zip(groups, sels):
        cz, cx, cy = ctrs[e][1:4]
        tconf, tcz, tcx, tcy = ctrs[e][4:8]
        t_valid = tconf > T_THR

        def match_terms(i, carry):
            mt, mp = carry
            dxi = i + 1
            for dz, dy in SHIFTS[1]:
                s = dz * Y + dy

                def SH(c):
                    return ch_scr[c, dxi, g, PAD_X:PAD_X + X,
                                  LANE0 + s:LANE0 + s + ZY]

                d2 = ((SH(1) - tcz) ** 2 + (SH(2) - tcx) ** 2
                      + (SH(3) - tcy) ** 2)
                ps = s2_scr[dxi, g, PAD_X:PAD_X + X, LANE0 + s:LANE0 + s + ZY]
                mt = jnp.maximum(mt, ((d2 < MATCH_R2) & (ps != 0))
                                 .astype(jnp.float32))
                d2b = ((SH(5) - cz) ** 2 + (SH(6) - cx) ** 2
                       + (SH(7) - cy) ** 2)
                mp = jnp.maximum(mp, ((d2b < MATCH_R2) & (SH(4) > T_THR))
                                 .astype(jnp.float32))
            return mt, mp

        mt_f, mp_f = jax.lax.fori_loop(
            0, 3, match_terms,
            (jnp.zeros((4, X, ZY), jnp.float32),
             jnp.zeros((4, X, ZY), jnp.float32)))
        matched_t = (mt_f != 0) & t_valid
        tp = count(matched_t)
        fn = count((mt_f == 0) & t_valid)
        fp = count(p_sel & (mp_f == 0))
        o_ref[g] = jnp.where(lane_id == 0, tp,
                             jnp.where(lane_id == 1, fp,
                                       jnp.where(lane_id == 2, fn, 0.0)))


def _prep(x):
    B = x.shape[0]
    v = x.reshape(B, 2, 4, Z, X, Y)
    v = jnp.transpose(v, (1, 0, 2, 4, 3, 5)).reshape(NP, 4, X, ZY)
    return jnp.pad(v, ((0, 0), (0, 0), (PAD_X, PAD_X), (LANE0, LANE0)),
                   constant_values=NEG)


@jax.jit
def kernel(predictions, targets):
    p = _prep(predictions)
    t = _prep(targets)
    out = pl.pallas_call(
        _nms_kernel,
        out_shape=jax.ShapeDtypeStruct((NP, 128), jnp.float32),
        scratch_shapes=[pltpu.VMEM((8, 5, NP, ROWS, LANES), jnp.float32),
                        pltpu.VMEM((NP, ROWS, LANES), jnp.float32),
                        pltpu.VMEM((NP, ROWS, LANES), jnp.float32),
                        pltpu.VMEM((5, NP, ROWS, LANES), jnp.float32),
                        pltpu.VMEM((5, NP, ROWS, LANES), jnp.float32),
                        pltpu.VMEM((5, NP, X, ZY), jnp.int32)],
    )(p, t)
    return jnp.transpose(out[:, :3].reshape(2, 4, 3), (1, 0, 2))
